# baseline (speedup 1.0000x reference)
import functools

import jax
import jax.numpy as jnp
from jax import lax
from jax.experimental import pallas as pl
from jax.experimental.pallas import tpu as pltpu
from jax.experimental.pallas import tpu_sc as plsc

_EMB = 128
_F32 = jnp.float32
_BPS = 4


def _sc_gather_rows(trans_pairs, tidx):
    info = plsc.get_sparse_core_info()
    nc, ns = info.num_cores, info.num_subcores
    nw = nc * ns
    n_off = tidx.shape[0]
    off_per_w = n_off // nw

    mesh = plsc.VectorSubcoreMesh(core_axis_name="c", subcore_axis_name="s")

    @functools.partial(
        pl.kernel,
        mesh=mesh,
        out_type=jax.ShapeDtypeStruct((n_off, 128), _F32),
        scratch_types=[
            pltpu.VMEM((off_per_w,), jnp.int32),
            pltpu.VMEM((off_per_w, 128), _F32),
            pltpu.SemaphoreType.DMA,
        ],
    )
    def body(trans_hbm, tidx_hbm, offrows_hbm, tidx_v, trows_v, sem):
        wid = lax.axis_index("s") * nc + lax.axis_index("c")
        tbase = wid * off_per_w
        pltpu.sync_copy(tidx_hbm.at[pl.ds(tbase, off_per_w)], tidx_v)
        pltpu.async_copy(trans_hbm.at[tidx_v], trows_v, sem).wait()
        pltpu.sync_copy(trows_v, offrows_hbm.at[pl.ds(tbase, off_per_w)])

    return body(trans_pairs, tidx)


def _node_norm_stats(x, n, ddof_n):
    s1 = jnp.sum(x, axis=1, keepdims=True)
    s2 = jnp.sum(x * x, axis=1, keepdims=True)
    mean = s1 / n
    var = (s2 - s1 * s1 / n) / ddof_n
    scale = 1.0 / (jnp.sqrt(var) + 1e-5)
    return mean, scale


def _node_norm_stats_t(x, n, ddof_n):
    s1 = jnp.sum(x, axis=0, keepdims=True)
    s2 = jnp.sum(x * x, axis=0, keepdims=True)
    mean = s1 / n
    var = (s2 - s1 * s1 / n) / ddof_n
    scale = 1.0 / (jnp.sqrt(var) + 1e-5)
    return mean, scale


def _half_select(rows128, par):
    return jnp.where(par > 0.5, rows128[:, 64:], rows128[:, :64])


def _tc_body(fo, fm, fv, pt, tt, orow, ostep, eob, vl, prev,
             wo, bo, wm, bm, wv, bv,
             eo_out, em_out, ev_out, pto_out, no_out, noff_out, nov_out,
             praw_s, ov_s, off_s, trans_s, e0o_s, e0m_s, e0v_s, acc_s):
    ph = pl.program_id(0)
    b = pl.program_id(1)
    nb = pl.num_programs(1)
    lane = lax.broadcasted_iota(jnp.int32, (1, 128), 1)

    @pl.when(ph == 0)
    def _phase0():
        vecs = None
        for i in range(_BPS):
            bi = _BPS * b + i
            x = fo[i]
            mean_o, scale_o = _node_norm_stats(x, 2000.0, 1999.0)
            step_v = jnp.minimum(ostep[pl.ds(bi, 1), :],
                                 eob[pl.ds(bi, 1), :])
            oh = (lax.broadcasted_iota(jnp.int32, (2000, 200), 0)
                  == step_v).astype(_F32)
            gat = lax.dot_general(x, oh, (((1,), (0,)), ((), ())))
            gn = (gat - mean_o) * scale_o
            eo = lax.dot_general(gn, wo[...],
                                 (((0,), (1,)), ((), ()))) + bo[...]
            eo_out[i] = eo

            xm = fm[i]
            mean_m, scale_m = _node_norm_stats_t(xm, 64.0, 63.0)
            nm = (xm - mean_m) * scale_m
            em = lax.dot_general(nm, wm[...],
                                 (((1,), (1,)), ((), ()))) + bm[...]
            em_out[i] = em

            xv = fv[i]
            mean_v, scale_v = _node_norm_stats_t(xv, 32.0, 31.0)
            nv = (xv - mean_v) * scale_v
            ev = lax.dot_general(nv, wv[...],
                                 (((1,), (1,)), ((), ()))) + bv[...]
            ev_out[i] = ev

            vrow = vl[pl.ds(bi, 1), :]
            eye = (lax.broadcasted_iota(jnp.int32, (32, 32), 0)
                   == lax.broadcasted_iota(jnp.int32, (32, 32), 1))
            par = lax.dot_general(eye.astype(_F32),
                                  (vrow & 1).astype(_F32),
                                  (((1,), (1,)), ((), ())))
            orows = _half_select(orow[i], par)
            prev_v = prev[pl.ds(bi, 1), :]
            ohp = (lax.broadcasted_iota(jnp.int32, (64, 200), 0)
                   == prev_v).astype(_F32)
            ov = lax.dot_general(orows, ohp,
                                 (((1,), (0,)), ((), ())))

            p = pt[i]
            praw = lax.dot_general(p, oh,
                                   (((1,), (0,)), ((), ())))

            t = tt[i]
            praw_s[_BPS * b + i] = praw
            ov_s[_BPS * b + i] = ov
            off_s[_BPS * b + i] = orows
            trans_s[_BPS * b + i] = t

            ps = jnp.sum(p)
            ps2 = jnp.sum(p * p)
            ts = jnp.sum(t)
            ts2 = jnp.sum(t * t)
            os_ = jnp.sum(orows)
            os2 = jnp.sum(orows * orows)
            vs = jnp.sum(ov)
            vs2 = jnp.sum(ov * ov)
            vec = jnp.where(lane == 0, ps, 0.0)
            vec = vec + jnp.where(lane == 1, ps2, 0.0)
            vec = vec + jnp.where(lane == 2, ts, 0.0)
            vec = vec + jnp.where(lane == 3, ts2, 0.0)
            vec = vec + jnp.where(lane == 4, os_, 0.0)
            vec = vec + jnp.where(lane == 5, os2, 0.0)
            vec = vec + jnp.where(lane == 6, vs, 0.0)
            vec = vec + jnp.where(lane == 7, vs2, 0.0)
            vecs = vec if vecs is None else vecs + vec

        @pl.when(b == 0)
        def _first():
            acc_s[...] = vecs
            e0o_s[...] = eo_out[...]
            e0m_s[...] = em_out[...]
            e0v_s[...] = ev_out[...]

        @pl.when(b != 0)
        def _rest():
            acc_s[...] = acc_s[...] + vecs

    @pl.when(ph == 1)
    def _phase1():
        S = acc_s[...]

        def pick(k):
            return jnp.sum(jnp.where(lane == k, S, 0.0))

        def mstats(s, s2, n):
            mean = s / n
            var = (s2 - s * s / n) / (n - 1.0)
            return mean, 1.0 / (jnp.sqrt(var) + 1e-5)

        m_p, sc_p = mstats(pick(0), pick(1), 8192000.0)
        m_t, sc_t = mstats(pick(2), pick(3), 262144.0)
        m_o, sc_o = mstats(pick(4), pick(5), 131072.0)
        m_v, sc_v = mstats(pick(6), pick(7), 409600.0)

        for i in range(_BPS):
            pto_out[i] = (praw_s[_BPS * b + i] - m_p) * sc_p
            no_out[i] = (trans_s[_BPS * b + i] - m_t) * sc_t
            noff_out[i] = (off_s[_BPS * b + i] - m_o) * sc_o
            nov_out[i] = (ov_s[_BPS * b + i] - m_v) * sc_v

        @pl.when(b == nb - 1)
        def _restore_first_embeds():
            eo_out[...] = e0o_s[...]
            em_out[...] = e0m_s[...]
            ev_out[...] = e0v_s[...]


def kernel(feat_opes_batch, feat_mas_batch, feat_vehs_batch,
           proc_times_batch, trans_times_batch, ope_ma_adj_batch,
           mask_job_finish_batch, mask_veh_procing_batch,
           mask_ma_procing_batch, ope_step_batch, end_ope_biases_batch,
           batch_idxes, veh_loc_batch, prev_ope_locs_batch, allo_ma_batch,
           W_opes, b_opes, W_mas, b_mas, W_vehs, b_vehs):
    B, ope_f, n_opes = feat_opes_batch.shape
    _, ma_f, n_mas = feat_mas_batch.shape
    _, veh_f, n_vehs = feat_vehs_batch.shape
    n_jobs = ope_step_batch.shape[1]

    vloc = veh_loc_batch.astype(jnp.int32)
    tboff = (jnp.arange(B, dtype=jnp.int32) * (n_mas // 2))[:, None]
    tidx = ((vloc >> 1) + tboff).reshape(-1)

    trans_flat = trans_times_batch.reshape(B * n_mas * n_mas // 128, 128)
    off_rows = _sc_gather_rows(trans_flat, tidx).reshape(B, n_vehs, 128)

    proc_t = jnp.transpose(proc_times_batch, (0, 2, 1))
    fm_t = jnp.transpose(feat_mas_batch, (0, 2, 1))
    fv_t = jnp.transpose(feat_vehs_batch, (0, 2, 1))

    ostep = ope_step_batch.astype(jnp.int32)
    eob = end_ope_biases_batch.astype(jnp.int32)
    prev2 = prev_ope_locs_batch.astype(jnp.int32)
    bo = b_opes.reshape(1, _EMB)
    bm = b_mas.reshape(1, _EMB)
    bv = b_vehs.reshape(1, _EMB)

    def p0spec(shape):
        return pl.BlockSpec((_BPS,) + shape, lambda p, b: (b * (1 - p), 0, 0))

    def p1spec(shape):
        return pl.BlockSpec((_BPS,) + shape, lambda p, b: (b * p, 0, 0))

    def fullspec(shape):
        nd = len(shape)
        return pl.BlockSpec(shape, lambda p, b, nd=nd: (0,) * nd)

    outs = pl.pallas_call(
        _tc_body,
        grid=(2, B // _BPS),
        in_specs=[
            p0spec((ope_f, n_opes)),
            p0spec((n_mas, ma_f)),
            p0spec((n_vehs, veh_f)),
            p0spec((n_mas, n_opes)),
            p0spec((n_mas, n_mas)),
            p0spec((n_vehs, 128)),
            fullspec((B, n_jobs)),
            fullspec((B, n_jobs)),
            fullspec((B, n_vehs)),
            fullspec((B, n_jobs)),
            fullspec((_EMB, ope_f)),
            fullspec((1, _EMB)),
            fullspec((_EMB, ma_f)),
            fullspec((1, _EMB)),
            fullspec((_EMB, veh_f)),
            fullspec((1, _EMB)),
        ],
        out_specs=[
            p0spec((n_jobs, _EMB)),
            p0spec((n_mas, _EMB)),
            p0spec((n_vehs, _EMB)),
            p1spec((n_mas, n_jobs)),
            p1spec((n_mas, n_mas)),
            p1spec((n_vehs, n_mas)),
            p1spec((n_vehs, n_jobs)),
        ],
        out_shape=[
            jax.ShapeDtypeStruct((B, n_jobs, _EMB), _F32),
            jax.ShapeDtypeStruct((B, n_mas, _EMB), _F32),
            jax.ShapeDtypeStruct((B, n_vehs, _EMB), _F32),
            jax.ShapeDtypeStruct((B, n_mas, n_jobs), _F32),
            jax.ShapeDtypeStruct((B, n_mas, n_mas), _F32),
            jax.ShapeDtypeStruct((B, n_vehs, n_mas), _F32),
            jax.ShapeDtypeStruct((B, n_vehs, n_jobs), _F32),
        ],
        scratch_shapes=[
            pltpu.VMEM((B, n_mas, n_jobs), _F32),
            pltpu.VMEM((B, n_vehs, n_jobs), _F32),
            pltpu.VMEM((B, n_vehs, n_mas), _F32),
            pltpu.VMEM((B, n_mas, n_mas), _F32),
            pltpu.VMEM((_BPS, n_jobs, _EMB), _F32),
            pltpu.VMEM((_BPS, n_mas, _EMB), _F32),
            pltpu.VMEM((_BPS, n_vehs, _EMB), _F32),
            pltpu.VMEM((1, _EMB), _F32),
        ],
    )(feat_opes_batch, fm_t, fv_t,
      proc_t, trans_times_batch, off_rows, ostep, eob, vloc, prev2,
      W_opes, bo, W_mas, bm, W_vehs, bv)

    (embed_opes, embed_mas, embed_vehs, pto_t,
     norm_onload, norm_offload, norm_offload_OV) = outs
    proc_time_out = jnp.transpose(pto_t, (0, 2, 1))
    return (embed_opes, embed_mas, embed_vehs, proc_time_out,
            norm_onload, norm_offload, norm_offload_OV)


# speedup vs baseline: 1.0083x; 1.0083x over previous
import functools

import jax
import jax.numpy as jnp
from jax import lax
from jax.experimental import pallas as pl
from jax.experimental.pallas import tpu as pltpu
from jax.experimental.pallas import tpu_sc as plsc

_EMB = 128
_F32 = jnp.float32
_BPS = 4


def _sc_gather_rows(trans_pairs, tidx):
    info = plsc.get_sparse_core_info()
    nc, ns = info.num_cores, info.num_subcores
    nw = nc * ns
    n_off = tidx.shape[0]
    off_per_w = n_off // nw

    mesh = plsc.VectorSubcoreMesh(core_axis_name="c", subcore_axis_name="s")

    @functools.partial(
        pl.kernel,
        mesh=mesh,
        out_type=jax.ShapeDtypeStruct((n_off, 128), _F32),
        scratch_types=[
            pltpu.VMEM((off_per_w,), jnp.int32),
            pltpu.VMEM((off_per_w, 128), _F32),
            pltpu.SemaphoreType.DMA,
        ],
    )
    def body(trans_hbm, tidx_hbm, offrows_hbm, tidx_v, trows_v, sem):
        wid = lax.axis_index("s") * nc + lax.axis_index("c")
        tbase = wid * off_per_w
        pltpu.sync_copy(tidx_hbm.at[pl.ds(tbase, off_per_w)], tidx_v)
        pltpu.async_copy(trans_hbm.at[tidx_v], trows_v, sem).wait()
        pltpu.sync_copy(trows_v, offrows_hbm.at[pl.ds(tbase, off_per_w)])

    return body(trans_pairs, tidx)


def _node_norm_stats(x, n, ddof_n):
    s1 = jnp.sum(x, axis=1, keepdims=True)
    s2 = jnp.sum(x * x, axis=1, keepdims=True)
    mean = s1 / n
    var = (s2 - s1 * s1 / n) / ddof_n
    scale = 1.0 / (jnp.sqrt(var) + 1e-5)
    return mean, scale


def _half_select(rows128, par):
    return jnp.where(par > 0.5, rows128[:, 64:], rows128[:, :64])


def _tc_body(fo, fm, fv, pt, tt, orow, opar, step, prev,
             wo, bo, wm, bm, wv, bv,
             eo_out, em_out, ev_out, pto_out, no_out, noff_out, nov_out,
             praw_s, ov_s, off_s, trans_s, e0o_s, e0m_s, e0v_s, acc_s):
    ph = pl.program_id(0)
    b = pl.program_id(1)
    nb = pl.num_programs(1)
    lane = lax.broadcasted_iota(jnp.int32, (1, 128), 1)

    @pl.when(ph == 0)
    def _phase0():
        vecs = None
        for i in range(_BPS):
            x = fo[i]
            mean_o, scale_o = _node_norm_stats(x, 2000.0, 1999.0)
            step_v = step[i]
            oh = (lax.broadcasted_iota(jnp.int32, (2000, 200), 0)
                  == step_v).astype(_F32)
            gat = lax.dot_general(x, oh, (((1,), (0,)), ((), ())))
            gn = (gat - mean_o) * scale_o
            eo = lax.dot_general(gn, wo[...],
                                 (((0,), (1,)), ((), ()))) + bo[...]
            eo_out[i] = eo

            xm = fm[i]
            mean_m, scale_m = _node_norm_stats(xm, 64.0, 63.0)
            nm = (xm - mean_m) * scale_m
            em = lax.dot_general(nm, wm[...],
                                 (((0,), (1,)), ((), ()))) + bm[...]
            em_out[i] = em

            xv = fv[i]
            mean_v, scale_v = _node_norm_stats(xv, 32.0, 31.0)
            nv = (xv - mean_v) * scale_v
            ev = lax.dot_general(nv, wv[...],
                                 (((0,), (1,)), ((), ()))) + bv[...]
            ev_out[i] = ev

            orows = _half_select(orow[i], opar[i])
            prev_v = prev[i]
            ohp = (lax.broadcasted_iota(jnp.int32, (64, 200), 0)
                   == prev_v).astype(_F32)
            ov = lax.dot_general(orows, ohp,
                                 (((1,), (0,)), ((), ())))

            p = pt[i]
            praw = lax.dot_general(p, oh,
                                   (((1,), (0,)), ((), ())))

            t = tt[i]
            praw_s[_BPS * b + i] = praw
            ov_s[_BPS * b + i] = ov
            off_s[_BPS * b + i] = orows
            trans_s[_BPS * b + i] = t

            ps = jnp.sum(p)
            ps2 = jnp.sum(p * p)
            ts = jnp.sum(t)
            ts2 = jnp.sum(t * t)
            os_ = jnp.sum(orows)
            os2 = jnp.sum(orows * orows)
            vs = jnp.sum(ov)
            vs2 = jnp.sum(ov * ov)
            vec = jnp.where(lane == 0, ps, 0.0)
            vec = vec + jnp.where(lane == 1, ps2, 0.0)
            vec = vec + jnp.where(lane == 2, ts, 0.0)
            vec = vec + jnp.where(lane == 3, ts2, 0.0)
            vec = vec + jnp.where(lane == 4, os_, 0.0)
            vec = vec + jnp.where(lane == 5, os2, 0.0)
            vec = vec + jnp.where(lane == 6, vs, 0.0)
            vec = vec + jnp.where(lane == 7, vs2, 0.0)
            vecs = vec if vecs is None else vecs + vec

        @pl.when(b == 0)
        def _first():
            acc_s[...] = vecs
            e0o_s[...] = eo_out[...]
            e0m_s[...] = em_out[...]
            e0v_s[...] = ev_out[...]

        @pl.when(b != 0)
        def _rest():
            acc_s[...] = acc_s[...] + vecs

    @pl.when(ph == 1)
    def _phase1():
        S = acc_s[...]

        def pick(k):
            return jnp.sum(jnp.where(lane == k, S, 0.0))

        def mstats(s, s2, n):
            mean = s / n
            var = (s2 - s * s / n) / (n - 1.0)
            return mean, 1.0 / (jnp.sqrt(var) + 1e-5)

        m_p, sc_p = mstats(pick(0), pick(1), 8192000.0)
        m_t, sc_t = mstats(pick(2), pick(3), 262144.0)
        m_o, sc_o = mstats(pick(4), pick(5), 131072.0)
        m_v, sc_v = mstats(pick(6), pick(7), 409600.0)

        for i in range(_BPS):
            pto_out[i] = (praw_s[_BPS * b + i] - m_p) * sc_p
            no_out[i] = (trans_s[_BPS * b + i] - m_t) * sc_t
            noff_out[i] = (off_s[_BPS * b + i] - m_o) * sc_o
            nov_out[i] = (ov_s[_BPS * b + i] - m_v) * sc_v

        @pl.when(b == nb - 1)
        def _restore_first_embeds():
            eo_out[...] = e0o_s[...]
            em_out[...] = e0m_s[...]
            ev_out[...] = e0v_s[...]


def kernel(feat_opes_batch, feat_mas_batch, feat_vehs_batch,
           proc_times_batch, trans_times_batch, ope_ma_adj_batch,
           mask_job_finish_batch, mask_veh_procing_batch,
           mask_ma_procing_batch, ope_step_batch, end_ope_biases_batch,
           batch_idxes, veh_loc_batch, prev_ope_locs_batch, allo_ma_batch,
           W_opes, b_opes, W_mas, b_mas, W_vehs, b_vehs):
    B, ope_f, n_opes = feat_opes_batch.shape
    _, ma_f, n_mas = feat_mas_batch.shape
    _, veh_f, n_vehs = feat_vehs_batch.shape
    n_jobs = ope_step_batch.shape[1]

    step = jnp.minimum(ope_step_batch, end_ope_biases_batch).astype(jnp.int32)
    vloc = veh_loc_batch.astype(jnp.int32)
    tboff = (jnp.arange(B, dtype=jnp.int32) * (n_mas // 2))[:, None]
    tidx = ((vloc >> 1) + tboff).reshape(-1)
    opar = (vloc & 1).astype(_F32).reshape(B, n_vehs, 1)

    trans_flat = trans_times_batch.reshape(B * n_mas * n_mas // 128, 128)
    off_rows = _sc_gather_rows(trans_flat, tidx).reshape(B, n_vehs, 128)

    proc_t = jnp.transpose(proc_times_batch, (0, 2, 1))

    step3 = step.reshape(B, 1, n_jobs)
    prev3 = prev_ope_locs_batch.astype(jnp.int32).reshape(B, 1, n_jobs)
    bo = b_opes.reshape(1, _EMB)
    bm = b_mas.reshape(1, _EMB)
    bv = b_vehs.reshape(1, _EMB)

    def p0spec(shape):
        return pl.BlockSpec((_BPS,) + shape, lambda p, b: (b * (1 - p), 0, 0))

    def p1spec(shape):
        return pl.BlockSpec((_BPS,) + shape, lambda p, b: (b * p, 0, 0))

    def fullspec(shape):
        nd = len(shape)
        return pl.BlockSpec(shape, lambda p, b, nd=nd: (0,) * nd)

    outs = pl.pallas_call(
        _tc_body,
        grid=(2, B // _BPS),
        in_specs=[
            p0spec((ope_f, n_opes)),
            p0spec((ma_f, n_mas)),
            p0spec((veh_f, n_vehs)),
            p0spec((n_mas, n_opes)),
            p0spec((n_mas, n_mas)),
            p0spec((n_vehs, 128)),
            p0spec((n_vehs, 1)),
            p0spec((1, n_jobs)),
            p0spec((1, n_jobs)),
            fullspec((_EMB, ope_f)),
            fullspec((1, _EMB)),
            fullspec((_EMB, ma_f)),
            fullspec((1, _EMB)),
            fullspec((_EMB, veh_f)),
            fullspec((1, _EMB)),
        ],
        out_specs=[
            p0spec((n_jobs, _EMB)),
            p0spec((n_mas, _EMB)),
            p0spec((n_vehs, _EMB)),
            p1spec((n_mas, n_jobs)),
            p1spec((n_mas, n_mas)),
            p1spec((n_vehs, n_mas)),
            p1spec((n_vehs, n_jobs)),
        ],
        out_shape=[
            jax.ShapeDtypeStruct((B, n_jobs, _EMB), _F32),
            jax.ShapeDtypeStruct((B, n_mas, _EMB), _F32),
            jax.ShapeDtypeStruct((B, n_vehs, _EMB), _F32),
            jax.ShapeDtypeStruct((B, n_mas, n_jobs), _F32),
            jax.ShapeDtypeStruct((B, n_mas, n_mas), _F32),
            jax.ShapeDtypeStruct((B, n_vehs, n_mas), _F32),
            jax.ShapeDtypeStruct((B, n_vehs, n_jobs), _F32),
        ],
        scratch_shapes=[
            pltpu.VMEM((B, n_mas, n_jobs), _F32),
            pltpu.VMEM((B, n_vehs, n_jobs), _F32),
            pltpu.VMEM((B, n_vehs, n_mas), _F32),
            pltpu.VMEM((B, n_mas, n_mas), _F32),
            pltpu.VMEM((_BPS, n_jobs, _EMB), _F32),
            pltpu.VMEM((_BPS, n_mas, _EMB), _F32),
            pltpu.VMEM((_BPS, n_vehs, _EMB), _F32),
            pltpu.VMEM((1, _EMB), _F32),
        ],
    )(feat_opes_batch, feat_mas_batch, feat_vehs_batch,
      proc_t, trans_times_batch, off_rows, opar, step3, prev3,
      W_opes, bo, W_mas, bm, W_vehs, bv)

    (embed_opes, embed_mas, embed_vehs, pto_t,
     norm_onload, norm_offload, norm_offload_OV) = outs
    proc_time_out = jnp.transpose(pto_t, (0, 2, 1))
    return (embed_opes, embed_mas, embed_vehs, proc_time_out,
            norm_onload, norm_offload, norm_offload_OV)
